# Initial kernel scaffold; baseline (speedup 1.0000x reference)
import jax
import jax.numpy as jnp
from jax.experimental import pallas as pl


def kernel(x, W1, b1, W2, b2, weight_main, U_all, S_all, V_all, bias):
    raise NotImplementedError("write your pallas kernel here")


# speedup vs baseline: 38.4156x; 38.4156x over previous
import functools

import jax
import jax.numpy as jnp
from jax.experimental import pallas as pl
from jax.experimental.pallas import tpu as pltpu

N_TOKENS = 8192
D_MODEL = 1024
HIDDEN = 256
NUM_EXPERTS = 8
TOP_K = 2
RANK = 64
ER = NUM_EXPERTS * RANK

BT = 256
GRID = N_TOKENS // BT


def _moe_block(x_ref, w1t_ref, b1_ref, w2t_ref, b2_ref, wmt_ref, vcat_ref,
               ucat_ref, sflat_ref, bias_ref, out_ref, loss_ref, acc_ref):
    i = pl.program_id(0)

    @pl.when(i == 0)
    def _init():
        acc_ref[...] = jnp.zeros_like(acc_ref)

    x = x_ref[...]

    h = jnp.maximum(
        jnp.dot(x, w1t_ref[...], preferred_element_type=jnp.float32)
        + b1_ref[...], 0.0)
    logits = (jnp.dot(h, w2t_ref[...], preferred_element_type=jnp.float32)
              + b2_ref[...])

    iota_e = jax.lax.broadcasted_iota(jnp.int32, (BT, NUM_EXPERTS), 1)
    m1 = jnp.max(logits, axis=1, keepdims=True)
    idx1 = jnp.min(jnp.where(logits == m1, iota_e, NUM_EXPERTS),
                   axis=1, keepdims=True)
    masked = jnp.where(iota_e == idx1, -1e30, logits)
    m2 = jnp.max(masked, axis=1, keepdims=True)
    idx2 = jnp.min(jnp.where(masked == m2, iota_e, NUM_EXPERTS),
                   axis=1, keepdims=True)

    e2 = jnp.exp(m2 - m1)
    g1 = 1.0 / (1.0 + e2)
    g2 = e2 * g1

    ex = jnp.exp(logits - m1)
    probs = ex / jnp.sum(ex, axis=1, keepdims=True)
    mask8 = ((iota_e == idx1) | (iota_e == idx2)).astype(jnp.float32)
    acc_ref[0:1, :] += jnp.sum(mask8, axis=0, keepdims=True)
    acc_ref[1:2, :] += jnp.sum(probs, axis=0, keepdims=True)

    e_of = jax.lax.broadcasted_iota(jnp.int32, (BT, ER), 1) // RANK
    w_exp = (g1 * (e_of == idx1).astype(jnp.float32)
             + g2 * (e_of == idx2).astype(jnp.float32))
    sw = w_exp * sflat_ref[...]

    xv = jnp.dot(x, vcat_ref[...], preferred_element_type=jnp.float32)
    expert = jnp.dot(xv * sw, ucat_ref[...],
                     preferred_element_type=jnp.float32)
    main = jnp.dot(x, wmt_ref[...], preferred_element_type=jnp.float32)
    out_ref[...] = main + expert + bias_ref[...]

    @pl.when(i == GRID - 1)
    def _finish():
        loss_ref[...] = (NUM_EXPERTS / (N_TOKENS * N_TOKENS)) * jnp.sum(
            acc_ref[0:1, :] * acc_ref[1:2, :], axis=(0, 1), keepdims=True)


@functools.partial(jax.jit, static_argnames=())
def kernel(x, W1, b1, W2, b2, weight_main, U_all, S_all, V_all, bias):
    w1t = W1.T
    w2t = W2.T
    wmt = weight_main.T
    vcat = V_all.reshape(ER, D_MODEL).T
    ucat = U_all.transpose(0, 2, 1).reshape(ER, D_MODEL)
    sflat = S_all.reshape(1, ER)
    b1r = b1.reshape(1, HIDDEN)
    b2r = b2.reshape(1, NUM_EXPERTS)
    biasr = bias.reshape(1, D_MODEL)

    const = lambda shape: pl.BlockSpec(shape, lambda i: (0, 0))
    out, loss = pl.pallas_call(
        _moe_block,
        grid=(GRID,),
        in_specs=[
            pl.BlockSpec((BT, D_MODEL), lambda i: (i, 0)),
            const((D_MODEL, HIDDEN)),
            const((1, HIDDEN)),
            const((HIDDEN, NUM_EXPERTS)),
            const((1, NUM_EXPERTS)),
            const((D_MODEL, D_MODEL)),
            const((D_MODEL, ER)),
            const((ER, D_MODEL)),
            const((1, ER)),
            const((1, D_MODEL)),
        ],
        out_specs=[
            pl.BlockSpec((BT, D_MODEL), lambda i: (i, 0)),
            pl.BlockSpec((1, 1), lambda i: (0, 0)),
        ],
        out_shape=[
            jax.ShapeDtypeStruct((N_TOKENS, D_MODEL), jnp.float32),
            jax.ShapeDtypeStruct((1, 1), jnp.float32),
        ],
        scratch_shapes=[pltpu.VMEM((2, NUM_EXPERTS), jnp.float32)],
        compiler_params=pltpu.CompilerParams(
            dimension_semantics=("arbitrary",)),
    )(x, w1t, b1r, w2t, b2r, wmt, vcat, ucat, sflat, biasr)
    return out, loss.reshape(())


# speedup vs baseline: 38.6719x; 1.0067x over previous
import functools

import jax
import jax.numpy as jnp
from jax.experimental import pallas as pl
from jax.experimental.pallas import tpu as pltpu

N_TOKENS = 8192
D_MODEL = 1024
HIDDEN = 256
NUM_EXPERTS = 8
TOP_K = 2
RANK = 64
ER = NUM_EXPERTS * RANK
ZW = D_MODEL + ER + HIDDEN

BT = 512
GRID = N_TOKENS // BT


def _moe_block(x_ref, wcat_ref, b1_ref, w2t_ref, b2_ref, exps_ref, ucat_ref,
               bias_ref, out_ref, loss_ref, acc_ref):
    i = pl.program_id(0)

    @pl.when(i == 0)
    def _init():
        acc_ref[...] = jnp.zeros_like(acc_ref)

    x = x_ref[...]

    z = jnp.dot(x, wcat_ref[...], preferred_element_type=jnp.float32)
    main = z[:, :D_MODEL]
    xv = z[:, D_MODEL:D_MODEL + ER]
    h = jnp.maximum(z[:, D_MODEL + ER:] + b1_ref[...], 0.0)

    logits = (jnp.dot(h.astype(jnp.bfloat16), w2t_ref[...],
                      preferred_element_type=jnp.float32)
              + b2_ref[...])

    iota_e = jax.lax.broadcasted_iota(jnp.int32, (BT, NUM_EXPERTS), 1)
    m1 = jnp.max(logits, axis=1, keepdims=True)
    idx1 = jnp.min(jnp.where(logits == m1, iota_e, NUM_EXPERTS),
                   axis=1, keepdims=True)
    masked = jnp.where(iota_e == idx1, -1e30, logits)
    m2 = jnp.max(masked, axis=1, keepdims=True)
    idx2 = jnp.min(jnp.where(masked == m2, iota_e, NUM_EXPERTS),
                   axis=1, keepdims=True)

    e2 = jnp.exp(m2 - m1)
    g1 = 1.0 / (1.0 + e2)
    g2 = e2 * g1

    ex = jnp.exp(logits - m1)
    probs = ex / jnp.sum(ex, axis=1, keepdims=True)
    sel1 = (iota_e == idx1).astype(jnp.float32)
    sel2 = (iota_e == idx2).astype(jnp.float32)
    acc_ref[0:1, :] += jnp.sum(sel1 + sel2, axis=0, keepdims=True)
    acc_ref[1:2, :] += jnp.sum(probs, axis=0, keepdims=True)

    w8 = g1 * sel1 + g2 * sel2
    sw = jnp.dot(w8.astype(jnp.bfloat16), exps_ref[...],
                 preferred_element_type=jnp.float32)

    expert = jnp.dot((xv * sw).astype(jnp.bfloat16), ucat_ref[...],
                     preferred_element_type=jnp.float32)
    out_ref[...] = main + expert + bias_ref[...]

    @pl.when(i == GRID - 1)
    def _finish():
        loss_ref[...] = (NUM_EXPERTS / (N_TOKENS * N_TOKENS)) * jnp.sum(
            acc_ref[0:1, :] * acc_ref[1:2, :], axis=(0, 1), keepdims=True)


@functools.partial(jax.jit, static_argnames=())
def kernel(x, W1, b1, W2, b2, weight_main, U_all, S_all, V_all, bias):
    bf = jnp.bfloat16
    vcat = V_all.reshape(ER, D_MODEL).T
    wcat = jnp.concatenate([weight_main.T, vcat, W1.T], axis=1).astype(bf)
    ucat = U_all.transpose(0, 2, 1).reshape(ER, D_MODEL).astype(bf)
    exps = (jnp.eye(NUM_EXPERTS, dtype=S_all.dtype)[:, :, None]
            * S_all[None, :, :]).reshape(NUM_EXPERTS, ER).astype(bf)
    xbf = x.astype(bf)
    w2t = W2.T.astype(bf)
    b1r = b1.reshape(1, HIDDEN)
    b2r = b2.reshape(1, NUM_EXPERTS)
    biasr = bias.reshape(1, D_MODEL)

    const = lambda shape: pl.BlockSpec(shape, lambda i: (0, 0))
    out, loss = pl.pallas_call(
        _moe_block,
        grid=(GRID,),
        in_specs=[
            pl.BlockSpec((BT, D_MODEL), lambda i: (i, 0)),
            const((D_MODEL, ZW)),
            const((1, HIDDEN)),
            const((HIDDEN, NUM_EXPERTS)),
            const((1, NUM_EXPERTS)),
            const((NUM_EXPERTS, ER)),
            const((ER, D_MODEL)),
            const((1, D_MODEL)),
        ],
        out_specs=[
            pl.BlockSpec((BT, D_MODEL), lambda i: (i, 0)),
            pl.BlockSpec((1, 1), lambda i: (0, 0)),
        ],
        out_shape=[
            jax.ShapeDtypeStruct((N_TOKENS, D_MODEL), jnp.float32),
            jax.ShapeDtypeStruct((1, 1), jnp.float32),
        ],
        scratch_shapes=[pltpu.VMEM((2, NUM_EXPERTS), jnp.float32)],
        compiler_params=pltpu.CompilerParams(
            dimension_semantics=("arbitrary",)),
    )(xbf, wcat, b1r, w2t, b2r, exps, ucat, biasr)
    return out, loss.reshape(())
